# Initial kernel scaffold; baseline (speedup 1.0000x reference)
import jax
import jax.numpy as jnp
from jax.experimental import pallas as pl


def kernel(embed1, edges, edge2, edge_id, W0, b0, bn_g, bn_b, gcn_W, gcn_b, diff_W, diff_b, fc1_W, fc1_b, fc2_W, fc2_b, fc3_W, fc3_b, cls_W, cls_b):
    raise NotImplementedError("write your pallas kernel here")


# speedup vs baseline: 12.0453x; 12.0453x over previous
import jax
import jax.numpy as jnp
from jax import lax
from jax.experimental import pallas as pl
from jax.experimental.pallas import tpu as pltpu
from jax.experimental.pallas import tpu_sc as plsc

N = 10000
E = 320000
C = 7
L = 2
D_IN = 128
H = 64
H2 = 2 * H
B = 16384

NC = 2
NS = 16
NT = NC * NS

NP = 10240
ET = E // NT
KC = 200
NCH = ET // KC
ROWS_PT = NP // NS
ZR = 128

GTOT = 2 * C * B
GT = GTOT // NT
GK = 512
GNCH = GT // GK

_MESH = plsc.VectorSubcoreMesh(core_axis_name="core", subcore_axis_name="sub")
_f32 = jnp.float32
_SC_PARAMS = pltpu.CompilerParams(needs_layout_passes=False)


def _count_body(esrc_ref, edst_ref, out_ref, idx_v, acc_v):
    core = lax.axis_index("core")
    sub = lax.axis_index("sub")
    wid = sub * NC + core
    ones = jnp.ones((16,), _f32)
    for c in range(C):
        def _zero(i, _):
            acc_v[pl.ds(i * 16, 16)] = jnp.zeros((16,), _f32)
            return 0
        lax.fori_loop(0, 2 * N // 16, _zero, 0)
        for side, src_ref in ((0, edst_ref), (1, esrc_ref)):
            base = c * E + wid * ET
            pltpu.sync_copy(src_ref.at[pl.ds(base, ET)], idx_v)

            def _scat(i, _):
                iv = idx_v[pl.ds(i * 16, 16)] + side * N
                plsc.addupdate_scatter(acc_v, [iv], ones)
                return 0

            lax.fori_loop(0, ET // 16, _scat, 0)
        pltpu.sync_copy(acc_v.at[pl.ds(0, N)], out_ref.at[core, sub, 2 * c, 0])
        pltpu.sync_copy(acc_v.at[pl.ds(N, N)],
                        out_ref.at[core, sub, 2 * c + 1, 0])


_count_call = pl.kernel(
    _count_body,
    out_type=jax.ShapeDtypeStruct((NC, NS, 2 * C, 1, N), _f32),
    mesh=_MESH,
    compiler_params=_SC_PARAMS,
    scratch_types=[
        pltpu.VMEM((ET,), jnp.int32),
        pltpu.VMEM((2 * N,), _f32),
    ],
)


def _segsum_body(xy_ref, eg_ref, es_ref, out_ref,
                 ig, isc, rows, zbuf, sem, acc):
    core = lax.axis_index("core")
    sub = lax.axis_index("sub")
    wid = sub * NC + core
    rbase = sub * ROWS_PT

    def _zb(i, _):
        for k in range(H2 // 16):
            zbuf[i, pl.ds(k * 16, 16)] = jnp.zeros((16,), _f32)
        return 0

    lax.fori_loop(0, ZR, _zb, 0)

    def _zero_slice():
        for q in range(ROWS_PT // ZR):
            pltpu.sync_copy(zbuf, acc.at[pl.ds(rbase + q * ZR, ZR)])

    _zero_slice()
    plsc.subcore_barrier()

    for c in range(C):
        cbase = c * E + wid * ET

        def _chunk(k, _):
            b0 = cbase + k * KC
            pltpu.sync_copy(eg_ref.at[pl.ds(b0, KC)], ig)
            pltpu.sync_copy(es_ref.at[pl.ds(b0, KC)], isc)
            pltpu.async_copy(xy_ref.at[ig], rows, sem).wait()
            pltpu.sync_copy(rows, acc.at[isc], add=True)
            return 0

        lax.fori_loop(0, NCH, _chunk, 0)
        plsc.subcore_barrier()
        pltpu.sync_copy(acc.at[pl.ds(rbase, ROWS_PT)], out_ref.at[core, c, sub])
        _zero_slice()
        plsc.subcore_barrier()


_segsum_call = pl.kernel(
    _segsum_body,
    out_type=jax.ShapeDtypeStruct((NC, C, NS, ROWS_PT, H2), _f32),
    mesh=_MESH,
    compiler_params=_SC_PARAMS,
    scratch_types=[
        pltpu.VMEM((KC,), jnp.int32),
        pltpu.VMEM((KC,), jnp.int32),
        pltpu.VMEM((KC, H2), _f32),
        pltpu.VMEM((ZR, H2), _f32),
        pltpu.SemaphoreType.DMA,
        pltpu.VMEM_SHARED((NP, H2), _f32),
    ],
)


def _gather_body(xy_ref, gidx_ref, g_out, idxv, rows, sem):
    core = lax.axis_index("core")
    sub = lax.axis_index("sub")
    wid = sub * NC + core
    base = wid * GT

    def _chunk(k, _):
        b0 = base + k * GK
        pltpu.sync_copy(gidx_ref.at[pl.ds(b0, GK)], idxv)
        pltpu.async_copy(xy_ref.at[idxv], rows, sem).wait()
        pltpu.sync_copy(rows, g_out.at[pl.ds(b0, GK)])
        return 0

    lax.fori_loop(0, GNCH, _chunk, 0)


_gather_call = pl.kernel(
    _gather_body,
    out_type=jax.ShapeDtypeStruct((GTOT, H2), _f32),
    mesh=_MESH,
    compiler_params=_SC_PARAMS,
    scratch_types=[
        pltpu.VMEM((GK,), jnp.int32),
        pltpu.VMEM((GK, H2), _f32),
        pltpu.SemaphoreType.DMA,
    ],
)


def _init_tc(e_ref, w_ref, b_ref, g_ref, bb_ref, cnts_ref,
             xy_ref, dinv_ref, cnt_ref):
    e = e_ref[...]
    t = jnp.dot(e, w_ref[0], preferred_element_type=_f32) + b_ref[0, 0]
    mu = jnp.mean(t, axis=0)
    var = jnp.mean((t - mu) ** 2, axis=0)
    xb = jnp.maximum(
        g_ref[0, 0] * (t - mu) * lax.rsqrt(var + 1e-5) + bb_ref[0, 0], 0.0)
    cc = cnts_ref[...]
    degm1 = jnp.sum(cc[:, :, 0, 0, :], axis=(0, 1))
    cnt = jnp.sum(cc[:, :, 0, 1, :], axis=(0, 1))
    dinv = lax.rsqrt(1.0 + degm1)
    xy_ref[0] = jnp.concatenate([xb, xb * dinv[:, None]], axis=1)
    dinv_ref[0] = dinv[:, None]
    cnt_ref[0] = cnt[:, None]


def _init_call(embed1, W0, b0, bn_g, bn_b, counts):
    return pl.pallas_call(
        _init_tc,
        grid=(C,),
        in_specs=[
            pl.BlockSpec((N, D_IN), lambda c: (0, 0)),
            pl.BlockSpec((1, D_IN, H), lambda c: (c, 0, 0)),
            pl.BlockSpec((1, 1, H), lambda c: (c, 0, 0)),
            pl.BlockSpec((1, 1, H), lambda c: (c, 0, 0)),
            pl.BlockSpec((1, 1, H), lambda c: (c, 0, 0)),
            pl.BlockSpec((NC, NS, 1, 2, N), lambda c: (0, 0, c, 0, 0)),
        ],
        out_specs=[
            pl.BlockSpec((1, N, H2), lambda c: (c, 0, 0)),
            pl.BlockSpec((1, N, 1), lambda c: (c, 0, 0)),
            pl.BlockSpec((1, N, 1), lambda c: (c, 0, 0)),
        ],
        out_shape=[
            jax.ShapeDtypeStruct((C, N, H2), _f32),
            jax.ShapeDtypeStruct((C, N, 1), _f32),
            jax.ShapeDtypeStruct((C, N, 1), _f32),
        ],
    )(embed1, W0, b0.reshape(C, 1, H), bn_g.reshape(C, 1, H),
      bn_b.reshape(C, 1, H), counts)


_BN = 2000


def _layer_tc(ap_ref, sp_ref, xy_in_ref, dinv_ref, cnt_ref,
              w_ref, b_ref, dw_ref, db_ref, xy_ref):
    x = xy_in_ref[0][:, :H]
    dinv = dinv_ref[0][:, :1]
    a = ap_ref[0, 0][:, H:] + ap_ref[1, 0][:, H:] + dinv * x
    h = jnp.maximum(
        dinv * jnp.dot(a, w_ref[0], preferred_element_type=_f32) + b_ref[0, 0],
        0.0)
    s = sp_ref[0, 0][:, :H] + sp_ref[1, 0][:, :H]
    msg = (jnp.dot(x * s, dw_ref[0], preferred_element_type=_f32)
           + cnt_ref[0][:, :1] * db_ref[0, 0])
    xo = h + msg
    xy_ref[0] = jnp.concatenate([xo, xo * dinv], axis=1)


def _layer_call(ap, sp, xy, dinv, cnt, w, b, dw, db):
    return pl.pallas_call(
        _layer_tc,
        grid=(C, N // _BN),
        in_specs=[
            pl.BlockSpec((NC, 1, _BN, H2), lambda c, r: (0, c, r, 0)),
            pl.BlockSpec((NC, 1, _BN, H2), lambda c, r: (0, c, r, 0)),
            pl.BlockSpec((1, _BN, H2), lambda c, r: (c, r, 0)),
            pl.BlockSpec((1, _BN, 1), lambda c, r: (c, r, 0)),
            pl.BlockSpec((1, _BN, 1), lambda c, r: (c, r, 0)),
            pl.BlockSpec((1, H, H), lambda c, r: (c, 0, 0)),
            pl.BlockSpec((1, 1, H), lambda c, r: (c, 0, 0)),
            pl.BlockSpec((1, H, H), lambda c, r: (c, 0, 0)),
            pl.BlockSpec((1, 1, H), lambda c, r: (c, 0, 0)),
        ],
        out_specs=pl.BlockSpec((1, _BN, H2), lambda c, r: (c, r, 0)),
        out_shape=jax.ShapeDtypeStruct((C, N, H2), _f32),
    )(ap, sp, xy, dinv, cnt, w, b.reshape(C, 1, H), dw, db.reshape(C, 1, H))


_BB = 2048


def _mlp_tc(g1_ref, g2_ref, f1w_ref, f1b_ref, f2w_ref, f2b_ref,
            f3w_ref, f3b_ref, cw_ref, cb_ref, o_ref):
    outs = []
    for c in range(C):
        a = g1_ref[c][:, :H]
        b2 = g2_ref[c][:, :H]
        f = (jnp.dot(a, f1w_ref[c, :H], preferred_element_type=_f32)
             + jnp.dot(b2, f1w_ref[c, H:2 * H], preferred_element_type=_f32)
             + jnp.dot(a * b2, f1w_ref[c, 2 * H:], preferred_element_type=_f32)
             + f1b_ref[c])
        h = jnp.maximum(f, 0.0)
        h = jnp.maximum(
            jnp.dot(h, f2w_ref[c], preferred_element_type=_f32) + f2b_ref[c],
            0.0)
        outs.append(
            jnp.dot(h, f3w_ref[c], preferred_element_type=_f32) + f3b_ref[c])
    z = jnp.maximum(jnp.concatenate(outs, axis=1), 0.0)
    o_ref[...] = jnp.dot(z, cw_ref[...], preferred_element_type=_f32) + cb_ref[...]


def _mlp_call(g1, g2, f1w, f1b, f2w, f2b, f3w, f3b, cw, cb):
    return pl.pallas_call(
        _mlp_tc,
        grid=(B // _BB,),
        in_specs=[
            pl.BlockSpec((C, _BB, H2), lambda i: (0, i, 0)),
            pl.BlockSpec((C, _BB, H2), lambda i: (0, i, 0)),
            pl.BlockSpec((C, 3 * H, 96), lambda i: (0, 0, 0)),
            pl.BlockSpec((C, 96), lambda i: (0, 0)),
            pl.BlockSpec((C, 96, 48), lambda i: (0, 0, 0)),
            pl.BlockSpec((C, 48), lambda i: (0, 0)),
            pl.BlockSpec((C, 48, 4), lambda i: (0, 0, 0)),
            pl.BlockSpec((C, 4), lambda i: (0, 0)),
            pl.BlockSpec((4 * C, C), lambda i: (0, 0)),
            pl.BlockSpec((C,), lambda i: (0,)),
        ],
        out_specs=pl.BlockSpec((_BB, C), lambda i: (i, 0)),
        out_shape=jax.ShapeDtypeStruct((B, C), _f32),
    )(g1, g2, f1w, f1b, f2w, f2b, f3w, f3b, cw, cb)


def kernel(embed1, edges, edge2, edge_id, W0, b0, bn_g, bn_b, gcn_W, gcn_b,
           diff_W, diff_b, fc1_W, fc1_b, fc2_W, fc2_b, fc3_W, fc3_b,
           cls_W, cls_b):
    edges = edges.astype(jnp.int32)
    offs = (jnp.arange(C, dtype=jnp.int32) * N)[:, None]
    esrc = edges[:, 0, :].reshape(-1)
    edst = edges[:, 1, :].reshape(-1)
    esrc_off = (edges[:, 0, :] + offs).reshape(-1)
    edst_off = (edges[:, 1, :] + offs).reshape(-1)

    counts = _count_call(esrc, edst).reshape(NC, NS, C, 2, N)
    xy, dinv, cnt = _init_call(embed1, W0, b0, bn_g, bn_b, counts)

    for j in range(L):
        xyf = xy.reshape(C * N, H2)
        ap = _segsum_call(xyf, esrc_off, edst).reshape(NC, C, NP, H2)
        sp = _segsum_call(xyf, edst_off, esrc).reshape(NC, C, NP, H2)
        xy = _layer_call(ap, sp, xy, dinv, cnt, gcn_W[:, j], gcn_b[:, j],
                         diff_W[:, j], diff_b[:, j])

    xyf = xy.reshape(C * N, H2)
    node_id = jnp.take(edge2, edge_id, axis=1).astype(jnp.int32)
    gidx = (node_id[:, None, :]
            + (jnp.arange(C, dtype=jnp.int32) * N)[None, :, None]).reshape(-1)
    g = _gather_call(xyf, gidx).reshape(2, C, B, H2)

    return _mlp_call(g[0], g[1], fc1_W, fc1_b, fc2_W, fc2_b,
                     fc3_W, fc3_b, cls_W, cls_b)
